# baseline (speedup 1.0000x reference)
import functools

import jax
import jax.numpy as jnp
from jax import lax
from jax.experimental import pallas as pl
from jax.experimental.pallas import tpu as pltpu
from jax.experimental.pallas import tpu_sc as plsc

N = 10000
E = 320000
D = 128

NC, NS = 2, 16
NW = NC * NS
EPW = E // NW
K = 80
NCHUNK = EPW // K
KA = 40
NCA = 256
NHALF = 2
HC = NCA // NHALF
EPAD = NCA * KA - EPW
NP = 10240
RPT = NP // NS

_sc_cache = {}


def _sc_mesh():
    if "mesh" not in _sc_cache:
        _sc_cache["mesh"] = plsc.VectorSubcoreMesh(
            core_axis_name="c", subcore_axis_name="s",
            num_cores=NC, num_subcores=NS)
    return _sc_cache["mesh"]


def _deg_body(src_hbm, dst_hbm, ones_hbm, zeros_hbm, out_hbm,
              src_v, dst_v, ones_v, acc_s, acc_d, sem):
    c = lax.axis_index("c")
    s = lax.axis_index("s")
    wid = c * NS + s
    pltpu.sync_copy(zeros_hbm, acc_s.at[pl.ds(s * RPT, RPT)])
    pltpu.sync_copy(zeros_hbm, acc_d.at[pl.ds(s * RPT, RPT)])
    pltpu.sync_copy(ones_hbm, ones_v)
    pltpu.sync_copy(src_hbm.at[wid], src_v)
    pltpu.sync_copy(dst_hbm.at[wid], dst_v)
    plsc.subcore_barrier()

    def chunk(i, carry):
        pltpu.sync_copy(ones_v, acc_s.at[src_v.at[i]], add=True)
        pltpu.sync_copy(ones_v, acc_d.at[dst_v.at[i]], add=True)
        return carry

    lax.fori_loop(0, NCHUNK, chunk, 0)
    plsc.subcore_barrier()
    pltpu.sync_copy(acc_s.at[pl.ds(s * RPT, RPT)],
                    out_hbm.at[0, c, 0, pl.ds(s * RPT, RPT)])
    pltpu.sync_copy(acc_d.at[pl.ds(s * RPT, RPT)],
                    out_hbm.at[1, c, 0, pl.ds(s * RPT, RPT)])


def _deg_kernel(*args):
    if "deg" not in _sc_cache:
        _sc_cache["deg"] = pl.kernel(
            _deg_body,
            out_type=jax.ShapeDtypeStruct((2, NC, 1, NP), jnp.float32),
            mesh=_sc_mesh(),
            scratch_types=[
                pltpu.VMEM((NCHUNK, K), jnp.int32),
                pltpu.VMEM((NCHUNK, K), jnp.int32),
                pltpu.VMEM((K,), jnp.float32),
                pltpu.VMEM_SHARED((NP,), jnp.float32),
                pltpu.VMEM_SHARED((NP,), jnp.float32),
                pltpu.SemaphoreType.DMA,
            ],
        )
    return _sc_cache["deg"](*args)


def _agg_body(table_hbm, src_hbm, dst_hbm, zeros_hbm, out_hbm,
              src_v, dst_v, rows0, acc, gsem0):
    c = lax.axis_index("c")
    s = lax.axis_index("s")
    wid = c * NS + s
    pltpu.sync_copy(zeros_hbm, acc.at[pl.ds(s * RPT, RPT)])
    plsc.subcore_barrier()

    for h in range(NHALF):
        pltpu.sync_copy(src_hbm.at[wid, pl.ds(h * HC, HC)], src_v)
        pltpu.sync_copy(dst_hbm.at[wid, pl.ds(h * HC, HC)], dst_v)

        def chunk(i, carry):
            pltpu.async_copy(table_hbm.at[src_v.at[i]], rows0, gsem0).wait()
            pltpu.sync_copy(rows0, acc.at[dst_v.at[i]], add=True)
            return carry

        lax.fori_loop(0, HC, chunk, 0)
    plsc.subcore_barrier()
    pltpu.sync_copy(acc.at[pl.ds(s * RPT, RPT)],
                    out_hbm.at[c, pl.ds(s * RPT, RPT)])


def _agg_kernel(*args):
    if "agg" not in _sc_cache:
        _sc_cache["agg"] = pl.kernel(
            _agg_body,
            out_type=jax.ShapeDtypeStruct((NC, NP, D), jnp.float32),
            mesh=_sc_mesh(),
            scratch_types=[
                pltpu.VMEM((HC, KA), jnp.int32),
                pltpu.VMEM((HC, KA), jnp.int32),
                pltpu.VMEM((KA, D), jnp.float32),
                pltpu.VMEM_SHARED((NP, D), jnp.float32),
                pltpu.SemaphoreType.DMA,
            ],
        )
    return _sc_cache["agg"](*args)


def _emb_body(nf_ref, wemb_ref, bemb_ref, w1_ref, x_ref, xw1_ref):
    x = jnp.dot(nf_ref[...], wemb_ref[...],
                preferred_element_type=jnp.float32) + bemb_ref[...]
    x_ref[...] = x
    xw1_ref[...] = jnp.dot(x, w1_ref[...], preferred_element_type=jnp.float32)


_emb_kernel = pl.pallas_call(
    _emb_body,
    out_shape=(
        jax.ShapeDtypeStruct((N, D), jnp.float32),
        jax.ShapeDtypeStruct((N, D), jnp.float32),
    ),
)


def _norm_body(degp_ref, xw1_ref, t1_ref, ns_ref, nd_ref):
    degp = degp_ref[...]
    ones_c = jnp.ones((NC, 1), jnp.float32)
    cdims = (((0,), (0,)), ((), ()))
    out_deg = lax.dot_general(degp[0, :, 0, :], ones_c, cdims,
                              preferred_element_type=jnp.float32)[:N]
    in_deg = lax.dot_general(degp[1, :, 0, :], ones_c, cdims,
                             preferred_element_type=jnp.float32)[:N]
    norm_src = jnp.where(out_deg > 0, lax.rsqrt(jnp.maximum(out_deg, 1.0)), 0.0)
    norm_dst = jnp.where(in_deg > 0, lax.rsqrt(jnp.maximum(in_deg, 1.0)), 0.0)
    t1_ref[...] = xw1_ref[...] * norm_src
    ns_ref[...] = norm_src
    nd_ref[...] = norm_dst


_norm_kernel = pl.pallas_call(
    _norm_body,
    out_shape=(
        jax.ShapeDtypeStruct((N, D), jnp.float32),
        jax.ShapeDtypeStruct((N, 1), jnp.float32),
        jax.ShapeDtypeStruct((N, 1), jnp.float32),
    ),
)


def _mid_body(x_ref, p_ref, nd_ref, b1_ref, gamma_ref, beta_ref,
              w2_ref, ns_ref, t2_ref):
    x = x_ref[...]
    p = p_ref[0, :N, :] + p_ref[1, :N, :]
    h = x + p * nd_ref[...] + b1_ref[...]
    mean = jnp.mean(h, axis=0, keepdims=True)
    var = jnp.mean((h - mean) ** 2, axis=0, keepdims=True)
    h = (h - mean) * lax.rsqrt(var + 1e-5) * gamma_ref[...] + beta_ref[...]
    h = jnp.maximum(h, 0.0)
    t2_ref[...] = jnp.dot(h, w2_ref[...],
                          preferred_element_type=jnp.float32) * ns_ref[...]


_mid_kernel = pl.pallas_call(
    _mid_body,
    out_shape=jax.ShapeDtypeStruct((N, D), jnp.float32),
)


def _fin_body(x_ref, p_ref, nd_ref, b2_ref, wc_ref, bc_ref, h_ref, lab_ref):
    h2 = x_ref[...] + (p_ref[0, :N, :] + p_ref[1, :N, :]) * nd_ref[...] + b2_ref[...]
    h_ref[...] = h2
    y = jnp.mean(h2, axis=0, keepdims=True)
    lab_ref[...] = jnp.dot(y, wc_ref[...],
                           preferred_element_type=jnp.float32) + bc_ref[...]


_fin_kernel = pl.pallas_call(
    _fin_body,
    out_shape=(
        jax.ShapeDtypeStruct((N, D), jnp.float32),
        jax.ShapeDtypeStruct((1, 60), jnp.float32),
    ),
)


def kernel(node_features, edge_index, W_emb, b_emb, W1, b1, gamma, beta,
           W2, b2, Wc, bc):
    src = edge_index[0].reshape(NW, NCHUNK, K)
    dst = edge_index[1].reshape(NW, NCHUNK, K)
    pad_rows = jnp.broadcast_to(N + jnp.arange(EPAD, dtype=jnp.int32),
                                (NW, EPAD))
    src_a = jnp.pad(edge_index[0].reshape(NW, EPW),
                    ((0, 0), (0, EPAD))).reshape(NW, NCA, KA)
    dst_a = jnp.concatenate(
        [edge_index[1].reshape(NW, EPW), pad_rows], axis=1
    ).reshape(NW, NCA, KA)
    onesK = jnp.ones((K,), jnp.float32)
    zerosR = jnp.zeros((RPT,), jnp.float32)
    zerosD = jnp.zeros((RPT, D), jnp.float32)

    degp = _deg_kernel(src, dst, onesK, zerosR)
    x, xw1 = _emb_kernel(node_features, W_emb, b_emb.reshape(1, D), W1)
    t1, norm_src, norm_dst = _norm_kernel(degp, xw1)
    p1 = _agg_kernel(t1, src_a, dst_a, zerosD)
    t2 = _mid_kernel(x, p1, norm_dst, b1.reshape(1, D), gamma.reshape(1, D),
                     beta.reshape(1, D), W2, norm_src)
    p2 = _agg_kernel(t2, src_a, dst_a, zerosD)
    h, lab = _fin_kernel(x, p2, norm_dst, b2.reshape(1, D), Wc,
                         bc.reshape(1, 60))
    return (h, lab.reshape(60))


# speedup vs baseline: 2.5573x; 2.5573x over previous
import functools

import jax
import jax.numpy as jnp
from jax import lax
from jax.experimental import pallas as pl
from jax.experimental.pallas import tpu as pltpu
from jax.experimental.pallas import tpu_sc as plsc

N = 10000
E = 320000
D = 128

NC, NS = 2, 16
NW = NC * NS
EPW = E // NW
K = 80
NCHUNK = EPW // K
KA = 100
NCA = 100
NHALF = 1
HC = NCA // NHALF
EPAD = NCA * KA - EPW
NP = 10240
RPT = NP // NS

_sc_cache = {}


def _sc_mesh():
    if "mesh" not in _sc_cache:
        _sc_cache["mesh"] = plsc.VectorSubcoreMesh(
            core_axis_name="c", subcore_axis_name="s",
            num_cores=NC, num_subcores=NS)
    return _sc_cache["mesh"]


def _deg_body(src_hbm, dst_hbm, ones_hbm, zeros_hbm, out_hbm,
              src_v, dst_v, ones_v, acc_s, acc_d, sem):
    c = lax.axis_index("c")
    s = lax.axis_index("s")
    wid = c * NS + s
    pltpu.sync_copy(zeros_hbm, acc_s.at[pl.ds(s * RPT, RPT)])
    pltpu.sync_copy(zeros_hbm, acc_d.at[pl.ds(s * RPT, RPT)])
    pltpu.sync_copy(ones_hbm, ones_v)
    pltpu.sync_copy(src_hbm.at[wid], src_v)
    pltpu.sync_copy(dst_hbm.at[wid], dst_v)
    plsc.subcore_barrier()

    def chunk(i, carry):
        pltpu.sync_copy(ones_v, acc_s.at[src_v.at[i]], add=True)
        pltpu.sync_copy(ones_v, acc_d.at[dst_v.at[i]], add=True)
        return carry

    lax.fori_loop(0, NCHUNK, chunk, 0)
    plsc.subcore_barrier()
    pltpu.sync_copy(acc_s.at[pl.ds(s * RPT, RPT)],
                    out_hbm.at[0, c, 0, pl.ds(s * RPT, RPT)])
    pltpu.sync_copy(acc_d.at[pl.ds(s * RPT, RPT)],
                    out_hbm.at[1, c, 0, pl.ds(s * RPT, RPT)])


def _deg_kernel(*args):
    if "deg" not in _sc_cache:
        _sc_cache["deg"] = pl.kernel(
            _deg_body,
            out_type=jax.ShapeDtypeStruct((2, NC, 1, NP), jnp.float32),
            mesh=_sc_mesh(),
            scratch_types=[
                pltpu.VMEM((NCHUNK, K), jnp.int32),
                pltpu.VMEM((NCHUNK, K), jnp.int32),
                pltpu.VMEM((K,), jnp.float32),
                pltpu.VMEM_SHARED((NP,), jnp.float32),
                pltpu.VMEM_SHARED((NP,), jnp.float32),
                pltpu.SemaphoreType.DMA,
            ],
        )
    return _sc_cache["deg"](*args)


def _agg_body(table_hbm, src_hbm, dst_hbm, zeros_hbm, out_hbm,
              src_v, dst_v, rows0, acc, gsem0):
    c = lax.axis_index("c")
    s = lax.axis_index("s")
    wid = c * NS + s
    pltpu.sync_copy(zeros_hbm, acc.at[pl.ds(s * RPT, RPT)])
    plsc.subcore_barrier()

    for h in range(NHALF):
        pltpu.sync_copy(src_hbm.at[wid, pl.ds(h * HC, HC)], src_v)
        pltpu.sync_copy(dst_hbm.at[wid, pl.ds(h * HC, HC)], dst_v)

        def chunk(i, carry):
            pltpu.async_copy(table_hbm.at[src_v.at[i]], rows0, gsem0).wait()
            pltpu.sync_copy(rows0, acc.at[dst_v.at[i]], add=True)
            return carry

        lax.fori_loop(0, HC, chunk, 0)
    plsc.subcore_barrier()
    pltpu.sync_copy(acc.at[pl.ds(s * RPT, RPT)],
                    out_hbm.at[c, pl.ds(s * RPT, RPT)])


def _agg_kernel(*args):
    if "agg" not in _sc_cache:
        _sc_cache["agg"] = pl.kernel(
            _agg_body,
            out_type=jax.ShapeDtypeStruct((NC, NP, D), jnp.float32),
            mesh=_sc_mesh(),
            scratch_types=[
                pltpu.VMEM((HC, KA), jnp.int32),
                pltpu.VMEM((HC, KA), jnp.int32),
                pltpu.VMEM((KA, D), jnp.float32),
                pltpu.VMEM_SHARED((NP, D), jnp.float32),
                pltpu.SemaphoreType.DMA,
            ],
        )
    return _sc_cache["agg"](*args)


def _emb_body(nf_ref, wemb_ref, bemb_ref, w1_ref, x_ref, xw1_ref):
    x = jnp.dot(nf_ref[...], wemb_ref[...],
                preferred_element_type=jnp.float32) + bemb_ref[...]
    x_ref[...] = x
    xw1_ref[...] = jnp.dot(x, w1_ref[...], preferred_element_type=jnp.float32)


_emb_kernel = pl.pallas_call(
    _emb_body,
    out_shape=(
        jax.ShapeDtypeStruct((N, D), jnp.float32),
        jax.ShapeDtypeStruct((N, D), jnp.float32),
    ),
)


def _norm_body(degp_ref, xw1_ref, t1_ref, ns_ref, nd_ref):
    degp = degp_ref[...]
    ones_c = jnp.ones((NC, 1), jnp.float32)
    cdims = (((0,), (0,)), ((), ()))
    out_deg = lax.dot_general(degp[0, :, 0, :], ones_c, cdims,
                              preferred_element_type=jnp.float32)[:N]
    in_deg = lax.dot_general(degp[1, :, 0, :], ones_c, cdims,
                             preferred_element_type=jnp.float32)[:N]
    norm_src = jnp.where(out_deg > 0, lax.rsqrt(jnp.maximum(out_deg, 1.0)), 0.0)
    norm_dst = jnp.where(in_deg > 0, lax.rsqrt(jnp.maximum(in_deg, 1.0)), 0.0)
    t1_ref[...] = xw1_ref[...] * norm_src
    ns_ref[...] = norm_src
    nd_ref[...] = norm_dst


_norm_kernel = pl.pallas_call(
    _norm_body,
    out_shape=(
        jax.ShapeDtypeStruct((N, D), jnp.float32),
        jax.ShapeDtypeStruct((N, 1), jnp.float32),
        jax.ShapeDtypeStruct((N, 1), jnp.float32),
    ),
)


def _mid_body(x_ref, p_ref, nd_ref, b1_ref, gamma_ref, beta_ref,
              w2_ref, ns_ref, t2_ref):
    x = x_ref[...]
    p = p_ref[0, :N, :] + p_ref[1, :N, :]
    h = x + p * nd_ref[...] + b1_ref[...]
    mean = jnp.mean(h, axis=0, keepdims=True)
    var = jnp.mean((h - mean) ** 2, axis=0, keepdims=True)
    h = (h - mean) * lax.rsqrt(var + 1e-5) * gamma_ref[...] + beta_ref[...]
    h = jnp.maximum(h, 0.0)
    t2_ref[...] = jnp.dot(h, w2_ref[...],
                          preferred_element_type=jnp.float32) * ns_ref[...]


_mid_kernel = pl.pallas_call(
    _mid_body,
    out_shape=jax.ShapeDtypeStruct((N, D), jnp.float32),
)


def _fin_body(x_ref, p_ref, nd_ref, b2_ref, wc_ref, bc_ref, h_ref, lab_ref):
    h2 = x_ref[...] + (p_ref[0, :N, :] + p_ref[1, :N, :]) * nd_ref[...] + b2_ref[...]
    h_ref[...] = h2
    y = jnp.mean(h2, axis=0, keepdims=True)
    lab_ref[...] = jnp.dot(y, wc_ref[...],
                           preferred_element_type=jnp.float32) + bc_ref[...]


_fin_kernel = pl.pallas_call(
    _fin_body,
    out_shape=(
        jax.ShapeDtypeStruct((N, D), jnp.float32),
        jax.ShapeDtypeStruct((1, 60), jnp.float32),
    ),
)


def kernel(node_features, edge_index, W_emb, b_emb, W1, b1, gamma, beta,
           W2, b2, Wc, bc):
    src = edge_index[0].reshape(NW, NCHUNK, K)
    dst = edge_index[1].reshape(NW, NCHUNK, K)
    src_a = edge_index[0].reshape(NW, NCA, KA)
    dst_a = edge_index[1].reshape(NW, NCA, KA)
    onesK = jnp.ones((K,), jnp.float32)
    zerosR = jnp.zeros((RPT,), jnp.float32)
    zerosD = jnp.zeros((RPT, D), jnp.float32)

    degp = _deg_kernel(src, dst, onesK, zerosR)
    x, xw1 = _emb_kernel(node_features, W_emb, b_emb.reshape(1, D), W1)
    t1, norm_src, norm_dst = _norm_kernel(degp, xw1)
    p1 = _agg_kernel(t1, src_a, dst_a, zerosD)
    t2 = _mid_kernel(x, p1, norm_dst, b1.reshape(1, D), gamma.reshape(1, D),
                     beta.reshape(1, D), W2, norm_src)
    p2 = _agg_kernel(t2, src_a, dst_a, zerosD)
    h, lab = _fin_kernel(x, p2, norm_dst, b2.reshape(1, D), Wc,
                         bc.reshape(1, 60))
    return (h, lab.reshape(60))


# speedup vs baseline: 2.7339x; 1.0691x over previous
import functools

import jax
import jax.numpy as jnp
from jax import lax
from jax.experimental import pallas as pl
from jax.experimental.pallas import tpu as pltpu
from jax.experimental.pallas import tpu_sc as plsc

N = 10000
E = 320000
D = 128

NC, NS = 2, 16
NW = NC * NS
EPW = E // NW
K = 80
NCHUNK = EPW // K
KA = 125
NCA = 80
NHALF = 1
HC = NCA // NHALF
EPAD = NCA * KA - EPW
NP = 10240
RPT = NP // NS

_sc_cache = {}


def _sc_mesh():
    if "mesh" not in _sc_cache:
        _sc_cache["mesh"] = plsc.VectorSubcoreMesh(
            core_axis_name="c", subcore_axis_name="s",
            num_cores=NC, num_subcores=NS)
    return _sc_cache["mesh"]


def _deg_body(src_hbm, dst_hbm, ones_hbm, zeros_hbm, out_hbm,
              src_v, dst_v, ones_v, acc_s, acc_d, sem):
    c = lax.axis_index("c")
    s = lax.axis_index("s")
    wid = c * NS + s
    pltpu.sync_copy(zeros_hbm, acc_s.at[pl.ds(s * RPT, RPT)])
    pltpu.sync_copy(zeros_hbm, acc_d.at[pl.ds(s * RPT, RPT)])
    pltpu.sync_copy(ones_hbm, ones_v)
    pltpu.sync_copy(src_hbm.at[wid], src_v)
    pltpu.sync_copy(dst_hbm.at[wid], dst_v)
    plsc.subcore_barrier()

    def chunk(i, carry):
        pltpu.sync_copy(ones_v, acc_s.at[src_v.at[i]], add=True)
        pltpu.sync_copy(ones_v, acc_d.at[dst_v.at[i]], add=True)
        return carry

    lax.fori_loop(0, NCHUNK, chunk, 0)
    plsc.subcore_barrier()
    pltpu.sync_copy(acc_s.at[pl.ds(s * RPT, RPT)],
                    out_hbm.at[0, c, 0, pl.ds(s * RPT, RPT)])
    pltpu.sync_copy(acc_d.at[pl.ds(s * RPT, RPT)],
                    out_hbm.at[1, c, 0, pl.ds(s * RPT, RPT)])


def _deg_kernel(*args):
    if "deg" not in _sc_cache:
        _sc_cache["deg"] = pl.kernel(
            _deg_body,
            out_type=jax.ShapeDtypeStruct((2, NC, 1, NP), jnp.float32),
            mesh=_sc_mesh(),
            scratch_types=[
                pltpu.VMEM((NCHUNK, K), jnp.int32),
                pltpu.VMEM((NCHUNK, K), jnp.int32),
                pltpu.VMEM((K,), jnp.float32),
                pltpu.VMEM_SHARED((NP,), jnp.float32),
                pltpu.VMEM_SHARED((NP,), jnp.float32),
                pltpu.SemaphoreType.DMA,
            ],
        )
    return _sc_cache["deg"](*args)


def _agg_body(table_hbm, src_hbm, dst_hbm, zeros_hbm, out_hbm,
              src_v, dst_v, rows0, acc, gsem0):
    c = lax.axis_index("c")
    s = lax.axis_index("s")
    wid = c * NS + s
    pltpu.sync_copy(zeros_hbm, acc.at[pl.ds(s * RPT, RPT)])
    plsc.subcore_barrier()

    for h in range(NHALF):
        pltpu.sync_copy(src_hbm.at[wid, pl.ds(h * HC, HC)], src_v)
        pltpu.sync_copy(dst_hbm.at[wid, pl.ds(h * HC, HC)], dst_v)

        def chunk(i, carry):
            pltpu.async_copy(table_hbm.at[src_v.at[i]], rows0, gsem0).wait()
            pltpu.sync_copy(rows0, acc.at[dst_v.at[i]], add=True)
            return carry

        lax.fori_loop(0, HC, chunk, 0)
    plsc.subcore_barrier()
    pltpu.sync_copy(acc.at[pl.ds(s * RPT, RPT)],
                    out_hbm.at[c, pl.ds(s * RPT, RPT)])


def _agg_kernel(*args):
    if "agg" not in _sc_cache:
        _sc_cache["agg"] = pl.kernel(
            _agg_body,
            out_type=jax.ShapeDtypeStruct((NC, NP, D), jnp.float32),
            mesh=_sc_mesh(),
            scratch_types=[
                pltpu.VMEM((HC, KA), jnp.int32),
                pltpu.VMEM((HC, KA), jnp.int32),
                pltpu.VMEM((KA, D), jnp.float32),
                pltpu.VMEM_SHARED((NP, D), jnp.float32),
                pltpu.SemaphoreType.DMA,
            ],
        )
    return _sc_cache["agg"](*args)


def _emb_body(nf_ref, wemb_ref, bemb_ref, w1_ref, x_ref, xw1_ref):
    x = jnp.dot(nf_ref[...], wemb_ref[...],
                preferred_element_type=jnp.float32) + bemb_ref[...]
    x_ref[...] = x
    xw1_ref[...] = jnp.dot(x, w1_ref[...], preferred_element_type=jnp.float32)


_emb_kernel = pl.pallas_call(
    _emb_body,
    out_shape=(
        jax.ShapeDtypeStruct((N, D), jnp.float32),
        jax.ShapeDtypeStruct((N, D), jnp.float32),
    ),
)


def _norm_body(degp_ref, xw1_ref, t1_ref, ns_ref, nd_ref):
    degp = degp_ref[...]
    ones_c = jnp.ones((NC, 1), jnp.float32)
    cdims = (((0,), (0,)), ((), ()))
    out_deg = lax.dot_general(degp[0, :, 0, :], ones_c, cdims,
                              preferred_element_type=jnp.float32)[:N]
    in_deg = lax.dot_general(degp[1, :, 0, :], ones_c, cdims,
                             preferred_element_type=jnp.float32)[:N]
    norm_src = jnp.where(out_deg > 0, lax.rsqrt(jnp.maximum(out_deg, 1.0)), 0.0)
    norm_dst = jnp.where(in_deg > 0, lax.rsqrt(jnp.maximum(in_deg, 1.0)), 0.0)
    t1_ref[...] = xw1_ref[...] * norm_src
    ns_ref[...] = norm_src
    nd_ref[...] = norm_dst


_norm_kernel = pl.pallas_call(
    _norm_body,
    out_shape=(
        jax.ShapeDtypeStruct((N, D), jnp.float32),
        jax.ShapeDtypeStruct((N, 1), jnp.float32),
        jax.ShapeDtypeStruct((N, 1), jnp.float32),
    ),
)


def _mid_body(x_ref, p_ref, nd_ref, b1_ref, gamma_ref, beta_ref,
              w2_ref, ns_ref, t2_ref):
    x = x_ref[...]
    p = p_ref[0, :N, :] + p_ref[1, :N, :]
    h = x + p * nd_ref[...] + b1_ref[...]
    mean = jnp.mean(h, axis=0, keepdims=True)
    var = jnp.mean((h - mean) ** 2, axis=0, keepdims=True)
    h = (h - mean) * lax.rsqrt(var + 1e-5) * gamma_ref[...] + beta_ref[...]
    h = jnp.maximum(h, 0.0)
    t2_ref[...] = jnp.dot(h, w2_ref[...],
                          preferred_element_type=jnp.float32) * ns_ref[...]


_mid_kernel = pl.pallas_call(
    _mid_body,
    out_shape=jax.ShapeDtypeStruct((N, D), jnp.float32),
)


def _fin_body(x_ref, p_ref, nd_ref, b2_ref, wc_ref, bc_ref, h_ref, lab_ref):
    h2 = x_ref[...] + (p_ref[0, :N, :] + p_ref[1, :N, :]) * nd_ref[...] + b2_ref[...]
    h_ref[...] = h2
    y = jnp.mean(h2, axis=0, keepdims=True)
    lab_ref[...] = jnp.dot(y, wc_ref[...],
                           preferred_element_type=jnp.float32) + bc_ref[...]


_fin_kernel = pl.pallas_call(
    _fin_body,
    out_shape=(
        jax.ShapeDtypeStruct((N, D), jnp.float32),
        jax.ShapeDtypeStruct((1, 60), jnp.float32),
    ),
)


def kernel(node_features, edge_index, W_emb, b_emb, W1, b1, gamma, beta,
           W2, b2, Wc, bc):
    src = edge_index[0].reshape(NW, NCHUNK, K)
    dst = edge_index[1].reshape(NW, NCHUNK, K)
    src_a = edge_index[0].reshape(NW, NCA, KA)
    dst_a = edge_index[1].reshape(NW, NCA, KA)
    onesK = jnp.ones((K,), jnp.float32)
    zerosR = jnp.zeros((RPT,), jnp.float32)
    zerosD = jnp.zeros((RPT, D), jnp.float32)

    degp = _deg_kernel(src, dst, onesK, zerosR)
    x, xw1 = _emb_kernel(node_features, W_emb, b_emb.reshape(1, D), W1)
    t1, norm_src, norm_dst = _norm_kernel(degp, xw1)
    p1 = _agg_kernel(t1, src_a, dst_a, zerosD)
    t2 = _mid_kernel(x, p1, norm_dst, b1.reshape(1, D), gamma.reshape(1, D),
                     beta.reshape(1, D), W2, norm_src)
    p2 = _agg_kernel(t2, src_a, dst_a, zerosD)
    h, lab = _fin_kernel(x, p2, norm_dst, b2.reshape(1, D), Wc,
                         bc.reshape(1, 60))
    return (h, lab.reshape(60))


# speedup vs baseline: 2.7422x; 1.0030x over previous
import functools

import jax
import jax.numpy as jnp
from jax import lax
from jax.experimental import pallas as pl
from jax.experimental.pallas import tpu as pltpu
from jax.experimental.pallas import tpu_sc as plsc

N = 10000
E = 320000
D = 128

NC, NS = 2, 16
NW = NC * NS
EPW = E // NW
K = 125
NCHUNK = EPW // K
KA = 125
NCA = 80
NHALF = 1
HC = NCA // NHALF
EPAD = NCA * KA - EPW
NP = 10240
RPT = NP // NS

_sc_cache = {}


def _sc_mesh():
    if "mesh" not in _sc_cache:
        _sc_cache["mesh"] = plsc.VectorSubcoreMesh(
            core_axis_name="c", subcore_axis_name="s",
            num_cores=NC, num_subcores=NS)
    return _sc_cache["mesh"]


def _deg_body(src_hbm, dst_hbm, ones_hbm, zeros_hbm, out_hbm,
              src_v, dst_v, ones_v, acc_s, acc_d, sem):
    c = lax.axis_index("c")
    s = lax.axis_index("s")
    wid = c * NS + s
    pltpu.sync_copy(zeros_hbm, acc_s.at[pl.ds(s * RPT, RPT)])
    pltpu.sync_copy(zeros_hbm, acc_d.at[pl.ds(s * RPT, RPT)])
    pltpu.sync_copy(ones_hbm, ones_v)
    pltpu.sync_copy(src_hbm.at[wid], src_v)
    pltpu.sync_copy(dst_hbm.at[wid], dst_v)
    plsc.subcore_barrier()

    def chunk(i, carry):
        pltpu.sync_copy(ones_v, acc_s.at[src_v.at[i]], add=True)
        pltpu.sync_copy(ones_v, acc_d.at[dst_v.at[i]], add=True)
        return carry

    lax.fori_loop(0, NCHUNK, chunk, 0)
    plsc.subcore_barrier()
    pltpu.sync_copy(acc_s.at[pl.ds(s * RPT, RPT)],
                    out_hbm.at[0, c, 0, pl.ds(s * RPT, RPT)])
    pltpu.sync_copy(acc_d.at[pl.ds(s * RPT, RPT)],
                    out_hbm.at[1, c, 0, pl.ds(s * RPT, RPT)])


def _deg_kernel(*args):
    if "deg" not in _sc_cache:
        _sc_cache["deg"] = pl.kernel(
            _deg_body,
            out_type=jax.ShapeDtypeStruct((2, NC, 1, NP), jnp.float32),
            mesh=_sc_mesh(),
            scratch_types=[
                pltpu.VMEM((NCHUNK, K), jnp.int32),
                pltpu.VMEM((NCHUNK, K), jnp.int32),
                pltpu.VMEM((K,), jnp.float32),
                pltpu.VMEM_SHARED((NP,), jnp.float32),
                pltpu.VMEM_SHARED((NP,), jnp.float32),
                pltpu.SemaphoreType.DMA,
            ],
        )
    return _sc_cache["deg"](*args)


def _agg_body(table_hbm, src_hbm, dst_hbm, zeros_hbm, out_hbm,
              src_v, dst_v, rows0, acc, gsem0):
    c = lax.axis_index("c")
    s = lax.axis_index("s")
    wid = c * NS + s
    pltpu.sync_copy(zeros_hbm, acc.at[pl.ds(s * RPT, RPT)])
    plsc.subcore_barrier()

    for h in range(NHALF):
        pltpu.sync_copy(src_hbm.at[wid, pl.ds(h * HC, HC)], src_v)
        pltpu.sync_copy(dst_hbm.at[wid, pl.ds(h * HC, HC)], dst_v)

        def chunk(i, carry):
            pltpu.async_copy(table_hbm.at[src_v.at[i]], rows0, gsem0).wait()
            pltpu.sync_copy(rows0, acc.at[dst_v.at[i]], add=True)
            return carry

        lax.fori_loop(0, HC, chunk, 0)
    plsc.subcore_barrier()
    pltpu.sync_copy(acc.at[pl.ds(s * RPT, RPT)],
                    out_hbm.at[c, pl.ds(s * RPT, RPT)])


def _agg_kernel(*args):
    if "agg" not in _sc_cache:
        _sc_cache["agg"] = pl.kernel(
            _agg_body,
            out_type=jax.ShapeDtypeStruct((NC, NP, D), jnp.float32),
            mesh=_sc_mesh(),
            scratch_types=[
                pltpu.VMEM((HC, KA), jnp.int32),
                pltpu.VMEM((HC, KA), jnp.int32),
                pltpu.VMEM((KA, D), jnp.float32),
                pltpu.VMEM_SHARED((NP, D), jnp.float32),
                pltpu.SemaphoreType.DMA,
            ],
        )
    return _sc_cache["agg"](*args)


def _emb_body(nf_ref, wemb_ref, bemb_ref, w1_ref, x_ref, xw1_ref):
    x = jnp.dot(nf_ref[...], wemb_ref[...],
                preferred_element_type=jnp.float32) + bemb_ref[...]
    x_ref[...] = x
    xw1_ref[...] = jnp.dot(x, w1_ref[...], preferred_element_type=jnp.float32)


_emb_kernel = pl.pallas_call(
    _emb_body,
    out_shape=(
        jax.ShapeDtypeStruct((N, D), jnp.float32),
        jax.ShapeDtypeStruct((N, D), jnp.float32),
    ),
)


def _norm_body(degp_ref, xw1_ref, t1_ref, ns_ref, nd_ref):
    degp = degp_ref[...]
    ones_c = jnp.ones((NC, 1), jnp.float32)
    cdims = (((0,), (0,)), ((), ()))
    out_deg = lax.dot_general(degp[0, :, 0, :], ones_c, cdims,
                              preferred_element_type=jnp.float32)[:N]
    in_deg = lax.dot_general(degp[1, :, 0, :], ones_c, cdims,
                             preferred_element_type=jnp.float32)[:N]
    norm_src = jnp.where(out_deg > 0, lax.rsqrt(jnp.maximum(out_deg, 1.0)), 0.0)
    norm_dst = jnp.where(in_deg > 0, lax.rsqrt(jnp.maximum(in_deg, 1.0)), 0.0)
    t1_ref[...] = xw1_ref[...] * norm_src
    ns_ref[...] = norm_src
    nd_ref[...] = norm_dst


_norm_kernel = pl.pallas_call(
    _norm_body,
    out_shape=(
        jax.ShapeDtypeStruct((N, D), jnp.float32),
        jax.ShapeDtypeStruct((N, 1), jnp.float32),
        jax.ShapeDtypeStruct((N, 1), jnp.float32),
    ),
)


def _mid_body(x_ref, p_ref, nd_ref, b1_ref, gamma_ref, beta_ref,
              w2_ref, ns_ref, t2_ref):
    x = x_ref[...]
    p = p_ref[0, :N, :] + p_ref[1, :N, :]
    h = x + p * nd_ref[...] + b1_ref[...]
    mean = jnp.mean(h, axis=0, keepdims=True)
    var = jnp.mean((h - mean) ** 2, axis=0, keepdims=True)
    h = (h - mean) * lax.rsqrt(var + 1e-5) * gamma_ref[...] + beta_ref[...]
    h = jnp.maximum(h, 0.0)
    t2_ref[...] = jnp.dot(h, w2_ref[...],
                          preferred_element_type=jnp.float32) * ns_ref[...]


_mid_kernel = pl.pallas_call(
    _mid_body,
    out_shape=jax.ShapeDtypeStruct((N, D), jnp.float32),
)


def _fin_body(x_ref, p_ref, nd_ref, b2_ref, wc_ref, bc_ref, h_ref, lab_ref):
    h2 = x_ref[...] + (p_ref[0, :N, :] + p_ref[1, :N, :]) * nd_ref[...] + b2_ref[...]
    h_ref[...] = h2
    y = jnp.mean(h2, axis=0, keepdims=True)
    lab_ref[...] = jnp.dot(y, wc_ref[...],
                           preferred_element_type=jnp.float32) + bc_ref[...]


_fin_kernel = pl.pallas_call(
    _fin_body,
    out_shape=(
        jax.ShapeDtypeStruct((N, D), jnp.float32),
        jax.ShapeDtypeStruct((1, 60), jnp.float32),
    ),
)


def kernel(node_features, edge_index, W_emb, b_emb, W1, b1, gamma, beta,
           W2, b2, Wc, bc):
    src = edge_index[0].reshape(NW, NCHUNK, K)
    dst = edge_index[1].reshape(NW, NCHUNK, K)
    src_a = edge_index[0].reshape(NW, NCA, KA)
    dst_a = edge_index[1].reshape(NW, NCA, KA)
    onesK = jnp.ones((K,), jnp.float32)
    zerosR = jnp.zeros((RPT,), jnp.float32)
    zerosD = jnp.zeros((RPT, D), jnp.float32)

    degp = _deg_kernel(src, dst, onesK, zerosR)
    x, xw1 = _emb_kernel(node_features, W_emb, b_emb.reshape(1, D), W1)
    t1, norm_src, norm_dst = _norm_kernel(degp, xw1)
    p1 = _agg_kernel(t1, src_a, dst_a, zerosD)
    t2 = _mid_kernel(x, p1, norm_dst, b1.reshape(1, D), gamma.reshape(1, D),
                     beta.reshape(1, D), W2, norm_src)
    p2 = _agg_kernel(t2, src_a, dst_a, zerosD)
    h, lab = _fin_kernel(x, p2, norm_dst, b2.reshape(1, D), Wc,
                         bc.reshape(1, 60))
    return (h, lab.reshape(60))
